# Initial kernel scaffold; baseline (speedup 1.0000x reference)
import jax
import jax.numpy as jnp
from jax.experimental import pallas as pl


def kernel(alpha, value, x_edge, node_pos, edge_dis, f_sparse_idx_node, rad_w1, rad_b1, rad_ln_g, rad_ln_b, rad_w2, rad_b2, so3_w, so3_b):
    raise NotImplementedError("write your pallas kernel here")


# speedup vs baseline: 1.2338x; 1.2338x over previous
import functools

import jax
import jax.numpy as jnp
from jax import lax
from jax.experimental import pallas as pl
from jax.experimental.pallas import tpu as pltpu
from jax.experimental.pallas import tpu_sc as plsc

N = 10000
K = 16
D = 128
NUM_COEF = 9
ROW = NUM_COEF * D

NW = 32
NPAD = 10240
PER_W = NPAD // NW
CHUNK = 4
NCHUNK = PER_W // CHUNK
EPAD = NPAD * K


def _stage_a_body(x_ref, al_ref, w1_ref, b1_ref, g_ref, bln_ref, w2_ref,
                  b2_ref, o_ref):
    x = x_ref[...]
    h = jnp.dot(x, w1_ref[...], preferred_element_type=jnp.float32)
    h = h + b1_ref[...]
    mu = jnp.mean(h, axis=-1, keepdims=True)
    var = jnp.mean((h - mu) ** 2, axis=-1, keepdims=True)
    h = (h - mu) * lax.rsqrt(var + 1e-5) * g_ref[...] + bln_ref[...]
    h = h * jax.nn.sigmoid(h)
    ih = jnp.dot(h, w2_ref[...], preferred_element_type=jnp.float32)
    ih = ih + b2_ref[...]
    hrow = lax.broadcasted_iota(jnp.int32, (8, D), 0)
    hcol = lax.broadcasted_iota(jnp.int32, (8, D), 1) // 16
    expand = (hrow == hcol).astype(jnp.float32)
    o_ref[...] = jnp.dot(al_ref[...], expand,
                         preferred_element_type=jnp.float32) * ih


def _stage_a(x2, al, w1, b1, g, bln, w2, b2):
    R = 2048
    grid = EPAD // R
    return pl.pallas_call(
        _stage_a_body,
        grid=(grid,),
        in_specs=[
            pl.BlockSpec((R, 16), lambda i: (i, 0)),
            pl.BlockSpec((R, 8), lambda i: (i, 0)),
            pl.BlockSpec((16, 64), lambda i: (0, 0)),
            pl.BlockSpec((1, 64), lambda i: (0, 0)),
            pl.BlockSpec((1, 64), lambda i: (0, 0)),
            pl.BlockSpec((1, 64), lambda i: (0, 0)),
            pl.BlockSpec((64, D), lambda i: (0, 0)),
            pl.BlockSpec((1, D), lambda i: (0, 0)),
        ],
        out_specs=pl.BlockSpec((R, D), lambda i: (i, 0)),
        out_shape=jax.ShapeDtypeStruct((EPAD, D), jnp.float32),
    )(x2, al, w1, b1, g, bln, w2, b2)


def _sc_gather_reduce(value2, idxf, a_pad):
    mesh = plsc.VectorSubcoreMesh(core_axis_name="c", subcore_axis_name="s")

    @functools.partial(
        pl.kernel,
        out_type=jax.ShapeDtypeStruct((NPAD, ROW), jnp.float32),
        mesh=mesh,
        scratch_types=[
            pltpu.VMEM((CHUNK * K,), jnp.int32),
            pltpu.VMEM((CHUNK * K, D), jnp.float32),
            pltpu.VMEM((CHUNK * K, ROW), jnp.float32),
            pltpu.VMEM((CHUNK, ROW), jnp.float32),
            pltpu.SemaphoreType.DMA,
        ],
    )
    def body(value_hbm, idx_hbm, a_hbm, out_hbm, idx_v, a_v, v_v, out_v, sem):
        wid = lax.axis_index("s") * 2 + lax.axis_index("c")

        def chunk_body(t, carry):
            nbase = wid * PER_W + t * CHUNK
            rbase = nbase * K
            pltpu.sync_copy(idx_hbm.at[pl.ds(rbase, CHUNK * K)], idx_v)
            pltpu.async_copy(value_hbm.at[idx_v], v_v, sem).wait()
            pltpu.sync_copy(a_hbm.at[pl.ds(rbase, CHUNK * K)], a_v)
            for i in range(CHUNK):
                for c8 in range(D // 16):
                    def k_body(k, accs, i=i, c8=c8):
                        r = i * K + k
                        av = a_v[r, pl.ds(c8 * 16, 16)]
                        return tuple(
                            accs[m] + v_v[r, pl.ds(m * D + c8 * 16, 16)] * av
                            for m in range(NUM_COEF))
                    accs = lax.fori_loop(
                        0, K, k_body,
                        tuple(jnp.zeros((16,), jnp.float32)
                              for _ in range(NUM_COEF)))
                    for m in range(NUM_COEF):
                        out_v[i, pl.ds(m * D + c8 * 16, 16)] = accs[m]
            pltpu.sync_copy(out_v, out_hbm.at[pl.ds(nbase, CHUNK)])
            return carry

        lax.fori_loop(0, NCHUNK, chunk_body, 0)

    return body(value2, idxf, a_pad)


def _stage_c_body(x_ref, w_ref, b_ref, o_ref):
    for m in range(NUM_COEF):
        l = 0 if m == 0 else (1 if m < 4 else 2)
        o = jnp.dot(x_ref[:, m, :], w_ref[l],
                    preferred_element_type=jnp.float32)
        if m == 0:
            o = o + b_ref[...]
        o_ref[:, m, :] = o


def _stage_c(x3, w, b):
    R = 512
    grid = NPAD // R
    return pl.pallas_call(
        _stage_c_body,
        grid=(grid,),
        in_specs=[
            pl.BlockSpec((R, NUM_COEF, D), lambda i: (i, 0, 0)),
            pl.BlockSpec((3, D, D), lambda i: (0, 0, 0)),
            pl.BlockSpec((1, D), lambda i: (0, 0)),
        ],
        out_specs=pl.BlockSpec((R, NUM_COEF, D), lambda i: (i, 0, 0)),
        out_shape=jax.ShapeDtypeStruct((NPAD, NUM_COEF, D), jnp.float32),
    )(x3, w, b)


def kernel(alpha, value, x_edge, node_pos, edge_dis, f_sparse_idx_node,
           rad_w1, rad_b1, rad_ln_g, rad_ln_b, rad_w2, rad_b2, so3_w, so3_b):
    ne = N * K
    x2 = jnp.pad(x_edge.reshape(ne, 16), ((0, EPAD - ne), (0, 0)))
    al = jnp.pad(alpha.reshape(ne, 8), ((0, EPAD - ne), (0, 0)))
    idxf = jnp.pad(f_sparse_idx_node.astype(jnp.int32).reshape(ne),
                   (0, EPAD - ne))
    value2 = value.reshape(N, ROW)

    a_pad = _stage_a(x2, al, rad_w1, rad_b1.reshape(1, 64),
                     rad_ln_g.reshape(1, 64), rad_ln_b.reshape(1, 64),
                     rad_w2, rad_b2.reshape(1, D))
    node_out = _sc_gather_reduce(value2, idxf, a_pad)
    out = _stage_c(node_out.reshape(NPAD, NUM_COEF, D), so3_w,
                   so3_b.reshape(1, D))
    return out[:N]


# speedup vs baseline: 1.5108x; 1.2245x over previous
import functools

import jax
import jax.numpy as jnp
from jax import lax
from jax.experimental import pallas as pl
from jax.experimental.pallas import tpu as pltpu
from jax.experimental.pallas import tpu_sc as plsc

N = 10000
K = 16
D = 128
NUM_COEF = 9
ROW = NUM_COEF * D

NW = 32
NPAD = 10240
PER_W = NPAD // NW
CHUNK = 2
NCHUNK = PER_W // CHUNK
EPAD = NPAD * K


def _stage_a_body(x_ref, al_ref, w1_ref, b1_ref, g_ref, bln_ref, w2_ref,
                  b2_ref, o_ref):
    x = x_ref[...]
    h = jnp.dot(x, w1_ref[...], preferred_element_type=jnp.float32)
    h = h + b1_ref[...]
    mu = jnp.mean(h, axis=-1, keepdims=True)
    var = jnp.mean((h - mu) ** 2, axis=-1, keepdims=True)
    h = (h - mu) * lax.rsqrt(var + 1e-5) * g_ref[...] + bln_ref[...]
    h = h * jax.nn.sigmoid(h)
    ih = jnp.dot(h, w2_ref[...], preferred_element_type=jnp.float32)
    ih = ih + b2_ref[...]
    hrow = lax.broadcasted_iota(jnp.int32, (8, D), 0)
    hcol = lax.broadcasted_iota(jnp.int32, (8, D), 1) // 16
    expand = (hrow == hcol).astype(jnp.float32)
    o_ref[...] = jnp.dot(al_ref[...], expand,
                         preferred_element_type=jnp.float32) * ih


def _stage_a(x2, al, w1, b1, g, bln, w2, b2):
    R = 2048
    grid = EPAD // R
    return pl.pallas_call(
        _stage_a_body,
        grid=(grid,),
        in_specs=[
            pl.BlockSpec((R, 16), lambda i: (i, 0)),
            pl.BlockSpec((R, 8), lambda i: (i, 0)),
            pl.BlockSpec((16, 64), lambda i: (0, 0)),
            pl.BlockSpec((1, 64), lambda i: (0, 0)),
            pl.BlockSpec((1, 64), lambda i: (0, 0)),
            pl.BlockSpec((1, 64), lambda i: (0, 0)),
            pl.BlockSpec((64, D), lambda i: (0, 0)),
            pl.BlockSpec((1, D), lambda i: (0, 0)),
        ],
        out_specs=pl.BlockSpec((R, D), lambda i: (i, 0)),
        out_shape=jax.ShapeDtypeStruct((EPAD, D), jnp.float32),
    )(x2, al, w1, b1, g, bln, w2, b2)


def _sc_gather_reduce(value2, idxf, a_pad):
    mesh = plsc.VectorSubcoreMesh(core_axis_name="c", subcore_axis_name="s")

    @functools.partial(
        pl.kernel,
        out_type=jax.ShapeDtypeStruct((NPAD, ROW), jnp.float32),
        mesh=mesh,
        scratch_types=[
            pltpu.VMEM((CHUNK * K,), jnp.int32),
            pltpu.VMEM((CHUNK * K,), jnp.int32),
            pltpu.VMEM((CHUNK * K, D), jnp.float32),
            pltpu.VMEM((CHUNK * K, D), jnp.float32),
            pltpu.VMEM((CHUNK * K, ROW), jnp.float32),
            pltpu.VMEM((CHUNK * K, ROW), jnp.float32),
            pltpu.VMEM((CHUNK, ROW), jnp.float32),
            pltpu.SemaphoreType.DMA,
            pltpu.SemaphoreType.DMA,
            pltpu.SemaphoreType.DMA,
            pltpu.SemaphoreType.DMA,
        ],
    )
    def body(value_hbm, idx_hbm, a_hbm, out_hbm,
             idx_v0, idx_v1, a_v0, a_v1, v_v0, v_v1, out_v,
             gsem0, gsem1, asem0, asem1):
        wid = lax.axis_index("s") * 2 + lax.axis_index("c")
        base = wid * PER_W
        idx_v = (idx_v0, idx_v1)
        a_v = (a_v0, a_v1)
        v_v = (v_v0, v_v1)
        gsem = (gsem0, gsem1)
        asem = (asem0, asem1)

        def fetch(b, t):
            rbase = (base + t * CHUNK) * K
            pltpu.sync_copy(idx_hbm.at[pl.ds(rbase, CHUNK * K)], idx_v[b])
            pltpu.async_copy(value_hbm.at[idx_v[b]], v_v[b], gsem[b])
            pltpu.async_copy(a_hbm.at[pl.ds(rbase, CHUNK * K)], a_v[b],
                             asem[b])

        def consume(b, t):
            pltpu.make_async_copy(value_hbm.at[idx_v[b]], v_v[b],
                                  gsem[b]).wait()
            pltpu.make_async_copy(a_hbm.at[pl.ds(0, CHUNK * K)], a_v[b],
                                  asem[b]).wait()
            for i in range(CHUNK):
                for c8 in range(D // 16):
                    def k_body(k, accs, i=i, c8=c8, b=b):
                        r = i * K + k
                        av = a_v[b][r, pl.ds(c8 * 16, 16)]
                        return tuple(
                            accs[m] + v_v[b][r, pl.ds(m * D + c8 * 16, 16)]
                            * av
                            for m in range(NUM_COEF))
                    accs = lax.fori_loop(
                        0, K, k_body,
                        tuple(jnp.zeros((16,), jnp.float32)
                              for _ in range(NUM_COEF)))
                    for m in range(NUM_COEF):
                        out_v[i, pl.ds(m * D + c8 * 16, 16)] = accs[m]
            pltpu.sync_copy(out_v, out_hbm.at[pl.ds(base + t * CHUNK, CHUNK)])

        fetch(0, 0)
        fetch(1, 1)

        def pair_body(p, carry):
            for b in range(2):
                t = 2 * p + b
                consume(b, t)
                fetch(b, t + 2)
            return carry

        lax.fori_loop(0, NCHUNK // 2 - 1, pair_body, 0)
        for b in range(2):
            consume(b, NCHUNK - 2 + b)

    return body(value2, idxf, a_pad)


def _stage_c_body(x_ref, w_ref, b_ref, o_ref):
    for m in range(NUM_COEF):
        l = 0 if m == 0 else (1 if m < 4 else 2)
        o = jnp.dot(x_ref[:, m, :], w_ref[l],
                    preferred_element_type=jnp.float32)
        if m == 0:
            o = o + b_ref[...]
        o_ref[:, m, :] = o


def _stage_c(x3, w, b):
    R = 512
    grid = NPAD // R
    return pl.pallas_call(
        _stage_c_body,
        grid=(grid,),
        in_specs=[
            pl.BlockSpec((R, NUM_COEF, D), lambda i: (i, 0, 0)),
            pl.BlockSpec((3, D, D), lambda i: (0, 0, 0)),
            pl.BlockSpec((1, D), lambda i: (0, 0)),
        ],
        out_specs=pl.BlockSpec((R, NUM_COEF, D), lambda i: (i, 0, 0)),
        out_shape=jax.ShapeDtypeStruct((NPAD, NUM_COEF, D), jnp.float32),
    )(x3, w, b)


def kernel(alpha, value, x_edge, node_pos, edge_dis, f_sparse_idx_node,
           rad_w1, rad_b1, rad_ln_g, rad_ln_b, rad_w2, rad_b2, so3_w, so3_b):
    ne = N * K
    x2 = jnp.pad(x_edge.reshape(ne, 16), ((0, EPAD - ne), (0, 0)))
    al = jnp.pad(alpha.reshape(ne, 8), ((0, EPAD - ne), (0, 0)))
    idxf = jnp.pad(f_sparse_idx_node.astype(jnp.int32).reshape(ne),
                   (0, EPAD - ne))
    value2 = value.reshape(N, ROW)

    a_pad = _stage_a(x2, al, rad_w1, rad_b1.reshape(1, 64),
                     rad_ln_g.reshape(1, 64), rad_ln_b.reshape(1, 64),
                     rad_w2, rad_b2.reshape(1, D))
    node_out = _sc_gather_reduce(value2, idxf, a_pad)
    out = _stage_c(node_out.reshape(NPAD, NUM_COEF, D), so3_w,
                   so3_b.reshape(1, D))
    return out[:N]


# speedup vs baseline: 1.9591x; 1.2967x over previous
import functools

import jax
import jax.numpy as jnp
from jax import lax
from jax.experimental import pallas as pl
from jax.experimental.pallas import tpu as pltpu
from jax.experimental.pallas import tpu_sc as plsc

N = 10000
K = 16
D = 128
NUM_COEF = 9
ROW = NUM_COEF * D

NW = 32
NPAD = 10240
PER_W = NPAD // NW
CHUNK = 2
NCHUNK = PER_W // CHUNK
NE = N * K


def _stage_a_body(x_ref, al_ref, w1_ref, b1_ref, g_ref, bln_ref, w2_ref,
                  b2_ref, o_ref):
    x = x_ref[...]
    h = jnp.dot(x, w1_ref[...], preferred_element_type=jnp.float32)
    h = h + b1_ref[...]
    mu = jnp.mean(h, axis=-1, keepdims=True)
    var = jnp.mean((h - mu) ** 2, axis=-1, keepdims=True)
    h = (h - mu) * lax.rsqrt(var + 1e-5) * g_ref[...] + bln_ref[...]
    h = h * jax.nn.sigmoid(h)
    ih = jnp.dot(h, w2_ref[...], preferred_element_type=jnp.float32)
    ih = ih + b2_ref[...]
    hrow = lax.broadcasted_iota(jnp.int32, (8, D), 0)
    hcol = lax.broadcasted_iota(jnp.int32, (8, D), 1) // 16
    expand = (hrow == hcol).astype(jnp.float32)
    o_ref[...] = jnp.dot(al_ref[...], expand,
                         preferred_element_type=jnp.float32) * ih


def _stage_a(x2, al, w1, b1, g, bln, w2, b2):
    R = 2000
    grid = NE // R
    return pl.pallas_call(
        _stage_a_body,
        grid=(grid,),
        in_specs=[
            pl.BlockSpec((R, 16), lambda i: (i, 0)),
            pl.BlockSpec((R, 8), lambda i: (i, 0)),
            pl.BlockSpec((16, 64), lambda i: (0, 0)),
            pl.BlockSpec((1, 64), lambda i: (0, 0)),
            pl.BlockSpec((1, 64), lambda i: (0, 0)),
            pl.BlockSpec((1, 64), lambda i: (0, 0)),
            pl.BlockSpec((64, D), lambda i: (0, 0)),
            pl.BlockSpec((1, D), lambda i: (0, 0)),
        ],
        out_specs=pl.BlockSpec((R, D), lambda i: (i, 0)),
        out_shape=jax.ShapeDtypeStruct((NE, D), jnp.float32),
    )(x2, al, w1, b1, g, bln, w2, b2)


def _sc_gather_reduce(value2, idxf, a_pad):
    mesh = plsc.VectorSubcoreMesh(core_axis_name="c", subcore_axis_name="s")

    @functools.partial(
        pl.kernel,
        out_type=jax.ShapeDtypeStruct((NPAD, ROW), jnp.float32),
        mesh=mesh,
        scratch_types=[
            pltpu.VMEM((CHUNK * K,), jnp.int32),
            pltpu.VMEM((CHUNK * K,), jnp.int32),
            pltpu.VMEM((CHUNK * K, D), jnp.float32),
            pltpu.VMEM((CHUNK * K, D), jnp.float32),
            pltpu.VMEM((CHUNK * K, ROW), jnp.float32),
            pltpu.VMEM((CHUNK * K, ROW), jnp.float32),
            pltpu.VMEM((CHUNK, ROW), jnp.float32),
            pltpu.SemaphoreType.DMA,
            pltpu.SemaphoreType.DMA,
            pltpu.SemaphoreType.DMA,
            pltpu.SemaphoreType.DMA,
        ],
    )
    def body(value_hbm, idx_hbm, a_hbm, out_hbm,
             idx_v0, idx_v1, a_v0, a_v1, v_v0, v_v1, out_v,
             gsem0, gsem1, asem0, asem1):
        wid = lax.axis_index("s") * 2 + lax.axis_index("c")
        base = wid * PER_W
        idx_v = (idx_v0, idx_v1)
        a_v = (a_v0, a_v1)
        v_v = (v_v0, v_v1)
        gsem = (gsem0, gsem1)
        asem = (asem0, asem1)

        def fetch(b, t):
            rbase = jnp.minimum(base + t * CHUNK, N - CHUNK) * K
            pltpu.sync_copy(idx_hbm.at[pl.ds(rbase, CHUNK * K)], idx_v[b])
            pltpu.async_copy(value_hbm.at[idx_v[b]], v_v[b], gsem[b])
            pltpu.async_copy(a_hbm.at[pl.ds(rbase, CHUNK * K)], a_v[b],
                             asem[b])

        def consume(b, t):
            pltpu.make_async_copy(value_hbm.at[idx_v[b]], v_v[b],
                                  gsem[b]).wait()
            pltpu.make_async_copy(a_hbm.at[pl.ds(0, CHUNK * K)], a_v[b],
                                  asem[b]).wait()
            for i in range(CHUNK):
                for c8 in range(D // 16):
                    def k_body(k, accs, i=i, c8=c8, b=b):
                        r = i * K + k
                        av = a_v[b][r, pl.ds(c8 * 16, 16)]
                        return tuple(
                            accs[m] + v_v[b][r, pl.ds(m * D + c8 * 16, 16)]
                            * av
                            for m in range(NUM_COEF))
                    accs = lax.fori_loop(
                        0, K, k_body,
                        tuple(jnp.zeros((16,), jnp.float32)
                              for _ in range(NUM_COEF)))
                    for m in range(NUM_COEF):
                        out_v[i, pl.ds(m * D + c8 * 16, 16)] = accs[m]
            pltpu.sync_copy(out_v, out_hbm.at[pl.ds(base + t * CHUNK, CHUNK)])

        fetch(0, 0)
        fetch(1, 1)

        def pair_body(p, carry):
            for b in range(2):
                t = 2 * p + b
                consume(b, t)
                fetch(b, t + 2)
            return carry

        lax.fori_loop(0, NCHUNK // 2 - 1, pair_body, 0)
        for b in range(2):
            consume(b, NCHUNK - 2 + b)

    return body(value2, idxf, a_pad)


def _stage_c_body(x_ref, w_ref, b_ref, o_ref):
    for m in range(NUM_COEF):
        l = 0 if m == 0 else (1 if m < 4 else 2)
        o = jnp.dot(x_ref[:, m, :], w_ref[l],
                    preferred_element_type=jnp.float32)
        if m == 0:
            o = o + b_ref[...]
        o_ref[:, m, :] = o


def _stage_c(x3, w, b):
    R = 512
    grid = NPAD // R
    return pl.pallas_call(
        _stage_c_body,
        grid=(grid,),
        in_specs=[
            pl.BlockSpec((R, NUM_COEF, D), lambda i: (i, 0, 0)),
            pl.BlockSpec((3, D, D), lambda i: (0, 0, 0)),
            pl.BlockSpec((1, D), lambda i: (0, 0)),
        ],
        out_specs=pl.BlockSpec((R, NUM_COEF, D), lambda i: (i, 0, 0)),
        out_shape=jax.ShapeDtypeStruct((NPAD, NUM_COEF, D), jnp.float32),
    )(x3, w, b)


def kernel(alpha, value, x_edge, node_pos, edge_dis, f_sparse_idx_node,
           rad_w1, rad_b1, rad_ln_g, rad_ln_b, rad_w2, rad_b2, so3_w, so3_b):
    x2 = x_edge.reshape(NE, 16)
    al = alpha.reshape(NE, 8)
    idxf = f_sparse_idx_node.astype(jnp.int32).reshape(NE)
    value2 = value.reshape(N, ROW)

    a_pad = _stage_a(x2, al, rad_w1, rad_b1.reshape(1, 64),
                     rad_ln_g.reshape(1, 64), rad_ln_b.reshape(1, 64),
                     rad_w2, rad_b2.reshape(1, D))
    node_out = _sc_gather_reduce(value2, idxf, a_pad)
    out = _stage_c(node_out.reshape(NPAD, NUM_COEF, D), so3_w,
                   so3_b.reshape(1, D))
    return out[:N]
